# baseline (speedup 1.0000x reference)
import functools

import jax
import jax.numpy as jnp
from jax.experimental import pallas as pl
from jax.experimental.pallas import tpu as pltpu

_BLK = 256


def _gcn_kernel(inter_ref, adj1_ref, adj2_ref, e1_ref, e2_ref,
                o1_ref, o2_ref, t1s_ref, *, half):
    i = pl.program_id(0)

    @pl.when(i % 2 == 0)
    def _():
        y1 = jnp.dot(adj1_ref[...], e1_ref[...],
                     preferred_element_type=jnp.float32)
        t1 = 4.0 * jnp.maximum(y1, 0.0)
        o1_ref[...] = t1
        t1s_ref[...] = t1

    @pl.when(i % 2 == 1)
    def _():
        y2 = jnp.dot(adj2_ref[...], e2_ref[...],
                     preferred_element_type=jnp.float32)
        t2 = 4.0 * jnp.maximum(y2, 0.0)
        j = i // 2

        @pl.when(j < half)
        def _():
            o2_ref[...] = t2

        @pl.when(j >= half)
        def _():
            w = inter_ref[0]
            o2_ref[...] = w * t1s_ref[...] + (1.0 - w) * t2


def kernel(adj1, adj2, dEmbed, mEmbed, pEmbed, inter):
    e1 = jnp.concatenate([dEmbed, mEmbed], axis=0)
    e2 = jnp.concatenate([pEmbed, mEmbed], axis=0)
    n = adj1.shape[0]
    d = dEmbed.shape[0]
    p = pEmbed.shape[0]
    f = dEmbed.shape[1]
    blk = _BLK
    nsteps = 2 * (n // blk)
    half = d // blk

    o1, o2 = pl.pallas_call(
        functools.partial(_gcn_kernel, half=half),
        grid=(nsteps,),
        in_specs=[
            pl.BlockSpec(memory_space=pltpu.SMEM),
            pl.BlockSpec((blk, n), lambda i: (i // 2, 0)),
            pl.BlockSpec((blk, n), lambda i: (jnp.maximum(i - 1, 0) // 2, 0)),
            pl.BlockSpec((n, f), lambda i: (0, 0)),
            pl.BlockSpec((n, f), lambda i: (0, 0)),
        ],
        out_specs=[
            pl.BlockSpec((blk, f), lambda i: (i // 2, 0)),
            pl.BlockSpec((blk, f), lambda i: (i // 2, 0)),
        ],
        out_shape=[
            jax.ShapeDtypeStruct((n, f), jnp.float32),
            jax.ShapeDtypeStruct((n, f), jnp.float32),
        ],
        scratch_shapes=[
            pltpu.VMEM((blk, f), jnp.float32),
        ],
    )(inter, adj1, adj2, e1, e2)
    return (o2[p:], o1[:d], o2[:p])


# speedup vs baseline: 1.1180x; 1.1180x over previous
import jax
import jax.numpy as jnp
from jax.experimental import pallas as pl
from jax.experimental.pallas import tpu as pltpu

_BLK = 128


def _gcn_kernel(inter_ref, a1lo_ref, a1hi_ref, a2lo_ref, a2hi_ref,
                de_ref, me_ref, pe_ref, d_ref, p_ref, m_ref):
    d = de_ref.shape[0]
    de = de_ref[...]
    me = me_ref[...]
    pe = pe_ref[...]

    y = (jnp.dot(a1lo_ref[:, :d], de, preferred_element_type=jnp.float32) +
         jnp.dot(a1lo_ref[:, d:], me, preferred_element_type=jnp.float32))
    d_ref[...] = 4.0 * jnp.maximum(y, 0.0)

    y = (jnp.dot(a2lo_ref[:, :d], pe, preferred_element_type=jnp.float32) +
         jnp.dot(a2lo_ref[:, d:], me, preferred_element_type=jnp.float32))
    p_ref[...] = 4.0 * jnp.maximum(y, 0.0)

    y = (jnp.dot(a1hi_ref[:, :d], de, preferred_element_type=jnp.float32) +
         jnp.dot(a1hi_ref[:, d:], me, preferred_element_type=jnp.float32))
    t1 = 4.0 * jnp.maximum(y, 0.0)
    y = (jnp.dot(a2hi_ref[:, :d], pe, preferred_element_type=jnp.float32) +
         jnp.dot(a2hi_ref[:, d:], me, preferred_element_type=jnp.float32))
    t2 = 4.0 * jnp.maximum(y, 0.0)
    w = inter_ref[0]
    m_ref[...] = w * t1 + (1.0 - w) * t2


def kernel(adj1, adj2, dEmbed, mEmbed, pEmbed, inter):
    n = adj1.shape[0]
    d = dEmbed.shape[0]
    m = mEmbed.shape[0]
    p = pEmbed.shape[0]
    f = dEmbed.shape[1]
    blk = _BLK
    grid = d // blk
    hoff = d // blk

    d_out, p_out, m_out = pl.pallas_call(
        _gcn_kernel,
        grid=(grid,),
        in_specs=[
            pl.BlockSpec(memory_space=pltpu.SMEM),
            pl.BlockSpec((blk, n), lambda i: (i, 0)),
            pl.BlockSpec((blk, n), lambda i: (i + hoff, 0)),
            pl.BlockSpec((blk, n), lambda i: (i, 0)),
            pl.BlockSpec((blk, n), lambda i: (i + hoff, 0)),
            pl.BlockSpec((d, f), lambda i: (0, 0)),
            pl.BlockSpec((m, f), lambda i: (0, 0)),
            pl.BlockSpec((p, f), lambda i: (0, 0)),
        ],
        out_specs=[
            pl.BlockSpec((blk, f), lambda i: (i, 0)),
            pl.BlockSpec((blk, f), lambda i: (i, 0)),
            pl.BlockSpec((blk, f), lambda i: (i, 0)),
        ],
        out_shape=[
            jax.ShapeDtypeStruct((d, f), jnp.float32),
            jax.ShapeDtypeStruct((p, f), jnp.float32),
            jax.ShapeDtypeStruct((m, f), jnp.float32),
        ],
    )(inter, adj1, adj1, adj2, adj2, dEmbed, mEmbed, pEmbed)
    return (m_out, d_out, p_out)


# speedup vs baseline: 1.1785x; 1.0541x over previous
import functools

import jax
import jax.numpy as jnp
from jax.experimental import pallas as pl
from jax.experimental.pallas import tpu as pltpu

_BLK = 256


def _gcn_kernel(inter_ref, adj1_ref, adj2_ref, de_ref, me_ref, pe_ref,
                o1_ref, o2_ref, *, half):
    i = pl.program_id(0)
    d = de_ref.shape[0]
    me = me_ref[...]
    y1 = (jnp.dot(adj1_ref[:, :d], de_ref[...],
                  preferred_element_type=jnp.float32) +
          jnp.dot(adj1_ref[:, d:], me, preferred_element_type=jnp.float32))
    y2 = (jnp.dot(adj2_ref[:, :d], pe_ref[...],
                  preferred_element_type=jnp.float32) +
          jnp.dot(adj2_ref[:, d:], me, preferred_element_type=jnp.float32))
    t1 = 4.0 * jnp.maximum(y1, 0.0)
    t2 = 4.0 * jnp.maximum(y2, 0.0)
    o1_ref[...] = t1

    @pl.when(i < half)
    def _():
        o2_ref[...] = t2

    @pl.when(i >= half)
    def _():
        w = inter_ref[0]
        o2_ref[...] = w * t1 + (1.0 - w) * t2


def kernel(adj1, adj2, dEmbed, mEmbed, pEmbed, inter):
    n = adj1.shape[0]
    d = dEmbed.shape[0]
    m = mEmbed.shape[0]
    p = pEmbed.shape[0]
    f = dEmbed.shape[1]
    blk = _BLK
    grid = n // blk
    half = d // blk

    o1, o2 = pl.pallas_call(
        functools.partial(_gcn_kernel, half=half),
        grid=(grid,),
        in_specs=[
            pl.BlockSpec(memory_space=pltpu.SMEM),
            pl.BlockSpec((blk, n), lambda i: (i, 0)),
            pl.BlockSpec((blk, n), lambda i: (i, 0)),
            pl.BlockSpec((d, f), lambda i: (0, 0)),
            pl.BlockSpec((m, f), lambda i: (0, 0)),
            pl.BlockSpec((p, f), lambda i: (0, 0)),
        ],
        out_specs=[
            pl.BlockSpec((blk, f), lambda i: (i, 0)),
            pl.BlockSpec((blk, f), lambda i: (i, 0)),
        ],
        out_shape=[
            jax.ShapeDtypeStruct((n, f), jnp.float32),
            jax.ShapeDtypeStruct((n, f), jnp.float32),
        ],
    )(inter, adj1, adj2, dEmbed, mEmbed, pEmbed)
    return (o2[p:], o1[:d], o2[:p])
